# Initial kernel scaffold; baseline (speedup 1.0000x reference)
import jax
import jax.numpy as jnp
from jax.experimental import pallas as pl


def kernel(scores, k):
    raise NotImplementedError("write your pallas kernel here")


# speedup vs baseline: 5.2565x; 5.2565x over previous
import jax
import jax.numpy as jnp
from jax import lax
from jax.experimental import pallas as pl
from jax.experimental.pallas import tpu as pltpu

_N = 32768
_B = 128
_ROWS = 16
_KMAX = 64


def _body(k_ref, x_ref, o_ref, keys_ref):
    x = x_ref[...]
    b = lax.bitcast_convert_type(x, jnp.int32)
    keys_ref[...] = jnp.where(b < 0, b ^ jnp.int32(0x7FFFFFFF), b)
    kk = k_ref[0]

    def vstep(j, lo):
        w = jnp.left_shift(jnp.int32(1), 31 - j)
        mid = lo + w
        cnt = jnp.sum((keys_ref[...] < mid).astype(jnp.int32), axis=1,
                      keepdims=True)
        return jnp.where(cnt >= kk, lo, mid)

    lo0 = jnp.full((_ROWS, 1), jnp.int32(-2147483648))
    t = lax.fori_loop(0, 32, vstep, lo0)

    s = keys_ref[...]
    c_less = jnp.sum((s < t).astype(jnp.int32), axis=1, keepdims=True)
    m = kk - c_less

    def istep(j, lo):
        w = jnp.left_shift(jnp.int32(1), 14 - j)
        mid = lo + w
        iota = lax.broadcasted_iota(jnp.int32, (_ROWS, _N), 1)
        cnt = jnp.sum(((keys_ref[...] == t) & (iota < mid)).astype(jnp.int32),
                      axis=1, keepdims=True)
        return jnp.where(cnt >= m, lo, mid)

    jcol = lax.fori_loop(0, 15, istep, jnp.zeros((_ROWS, 1), jnp.int32))

    iota = lax.broadcasted_iota(jnp.int32, (_ROWS, _N), 1)
    keep = ((s < t) | ((s == t) & (iota <= jcol))) & (kk > 0)
    o_ref[...] = jnp.where(keep, x, jnp.float32(-jnp.inf))


def kernel(scores, k):
    kk = jnp.clip(jnp.asarray(k, jnp.int32), 0, _KMAX).reshape((1,))
    return pl.pallas_call(
        _body,
        grid=(_B // _ROWS,),
        in_specs=[
            pl.BlockSpec(memory_space=pltpu.SMEM),
            pl.BlockSpec((_ROWS, _N), lambda i: (i, 0)),
        ],
        out_specs=pl.BlockSpec((_ROWS, _N), lambda i: (i, 0)),
        out_shape=jax.ShapeDtypeStruct(scores.shape, scores.dtype),
        scratch_shapes=[pltpu.VMEM((_ROWS, _N), jnp.int32)],
    )(kk, scores)
